# Initial kernel scaffold; baseline (speedup 1.0000x reference)
import jax
import jax.numpy as jnp
from jax.experimental import pallas as pl


def kernel(hidden_states, router_w, correction_bias, w_gate_up, w_down):
    raise NotImplementedError("write your pallas kernel here")


# speedup vs baseline: 1.9176x; 1.9176x over previous
import jax
import jax.numpy as jnp
from jax.experimental import pallas as pl
from jax.experimental.pallas import tpu as pltpu

E = 8
TOPK = 2
D = 1024
DFF = 512
T = 2048
BT = 256
NTB = T // BT


def _router_body(x_ref, rw_ref, cb_ref, comb_ref):
    x = x_ref[...]
    logits = jnp.dot(x, rw_ref[...], preferred_element_type=jnp.float32)
    m = jnp.max(logits, axis=-1, keepdims=True)
    ex = jnp.exp(logits - m)
    scores = ex / jnp.sum(ex, axis=-1, keepdims=True)
    b = scores + cb_ref[...]
    ids = jax.lax.broadcasted_iota(jnp.int32, (T, E), 1)
    m1 = jnp.max(b, axis=-1, keepdims=True)
    i1 = jnp.min(jnp.where(b == m1, ids, E), axis=-1, keepdims=True)
    b2 = jnp.where(ids == i1, -1e30, b)
    m2 = jnp.max(b2, axis=-1, keepdims=True)
    i2 = jnp.min(jnp.where(b2 == m2, ids, E), axis=-1, keepdims=True)
    w1 = jnp.sum(jnp.where(ids == i1, scores, 0.0), axis=-1, keepdims=True)
    w2 = jnp.sum(jnp.where(ids == i2, scores, 0.0), axis=-1, keepdims=True)
    comb_ref[...] = jnp.where(ids == i1, w1, 0.0) + jnp.where(ids == i2, w2, 0.0)


def _moe_body(comb_ref, x_ref, wgu_ref, wd_ref, out_ref):
    e = pl.program_id(0)
    tb = pl.program_id(1)
    row0 = pl.multiple_of(tb * BT, BT)
    x = x_ref[pl.ds(row0, BT), :].astype(jnp.bfloat16)
    wgu = wgu_ref[0].astype(jnp.bfloat16)
    gu = jnp.dot(x, wgu, preferred_element_type=jnp.float32)
    gate = gu[:, :DFF]
    up = gu[:, DFF:]
    h = (gate * jax.lax.logistic(gate) * up).astype(jnp.bfloat16)
    wd = wd_ref[0].astype(jnp.bfloat16)
    y = jnp.dot(h, wd, preferred_element_type=jnp.float32)
    cslice = comb_ref[pl.ds(row0, BT), :]
    c = jnp.zeros((BT, 1), jnp.float32)
    for j in range(E):
        c = c + jnp.where(e == j, cslice[:, j:j + 1], 0.0)
    contrib = y * c

    @pl.when(e == 0)
    def _init():
        out_ref[pl.ds(row0, BT), :] = contrib

    @pl.when(e != 0)
    def _acc():
        out_ref[pl.ds(row0, BT), :] += contrib


def kernel(hidden_states, router_w, correction_bias, w_gate_up, w_down):
    cb2 = correction_bias.reshape(1, E)
    comb = pl.pallas_call(
        _router_body,
        out_shape=jax.ShapeDtypeStruct((T, E), jnp.float32),
    )(hidden_states, router_w, cb2)

    out = pl.pallas_call(
        _moe_body,
        grid=(E, NTB),
        in_specs=[
            pl.BlockSpec((T, E), lambda e, tb: (0, 0)),
            pl.BlockSpec((T, D), lambda e, tb: (0, 0)),
            pl.BlockSpec((1, D, 2 * DFF), lambda e, tb: (e, 0, 0)),
            pl.BlockSpec((1, DFF, D), lambda e, tb: (e, 0, 0)),
        ],
        out_specs=pl.BlockSpec((T, D), lambda e, tb: (0, 0)),
        out_shape=jax.ShapeDtypeStruct((T, D), jnp.float32),
    )(comb, hidden_states, w_gate_up, w_down)
    return out
